# Initial kernel scaffold; baseline (speedup 1.0000x reference)
import jax
import jax.numpy as jnp
from jax.experimental import pallas as pl


def kernel(atom_feats, bond_feats, global_feats, atom_segment_ids, bond_segment_ids):
    raise NotImplementedError("write your pallas kernel here")


# speedup vs baseline: 2.8570x; 2.8570x over previous
import functools

import jax
import jax.numpy as jnp
from jax.experimental import pallas as pl

B = 512
D = 128


def _seg_sum_body(ids_ref, x_ref, o_ref, *, rows):
    i = pl.program_id(0)

    @pl.when(i == 0)
    def _():
        o_ref[...] = jnp.zeros_like(o_ref)

    ids = ids_ref[0, :, :]
    x = x_ref[:, 0, 0, :]
    iota = jax.lax.broadcasted_iota(jnp.int32, (B, rows), 0)
    m = (iota == ids).astype(jnp.float32)
    o_ref[...] += jnp.dot(m, x, preferred_element_type=jnp.float32)


def _segment_sum(x4, ids, rows):
    n = x4.shape[0]
    nblocks = n // rows
    ids3 = ids.reshape(nblocks, 1, rows)
    return pl.pallas_call(
        functools.partial(_seg_sum_body, rows=rows),
        grid=(nblocks,),
        in_specs=[
            pl.BlockSpec((1, 1, rows), lambda i: (i, 0, 0)),
            pl.BlockSpec((rows, 1, 1, D), lambda i: (i, 0, 0, 0)),
        ],
        out_specs=pl.BlockSpec((B, D), lambda i: (0, 0)),
        out_shape=jax.ShapeDtypeStruct((B, D), jnp.float32),
    )(ids3, x4)


def kernel(atom_feats, bond_feats, global_feats, atom_segment_ids, bond_segment_ids):
    a_ids = atom_segment_ids.astype(jnp.int32)
    b_ids = bond_segment_ids.astype(jnp.int32)
    n_atoms = atom_feats.shape[0]
    n_bonds = bond_feats.shape[0] // 2

    atoms4 = atom_feats.reshape(n_atoms, 1, 1, D)
    bonds4 = bond_feats.reshape(n_bonds, 2, 1, D)

    atom_pooled = _segment_sum(atoms4, a_ids, rows=400)
    bond_pooled = _segment_sum(bonds4, b_ids, rows=320)
    return jnp.concatenate([atom_pooled, bond_pooled, global_feats], axis=-1)


# speedup vs baseline: 6.7378x; 2.3584x over previous
import functools

import jax
import jax.numpy as jnp
from jax import lax
from jax.experimental import pallas as pl
from jax.experimental.pallas import tpu as pltpu
from jax.experimental.pallas import tpu_sc as plsc

B = 512
D = 128

N_ATOMS = 10000
N_BONDS = 160000

BBLK = 128
ABLK = 80
NB_BOND = N_BONDS // BBLK
NB_ATOM = N_ATOMS // ABLK
NW = 32


def _sc_pool_body(
    bond_hbm,
    evidx_hbm,
    bseg_hbm,
    atom_hbm,
    aseg_hbm,
    out_a_hbm,
    out_b_hbm,
    acc_a,
    acc_b,
    rows_v,
    arows_v,
    idx_v,
    bseg_v,
    aseg_v,
    tmp_v,
):
    cid = lax.axis_index("c")
    sid = lax.axis_index("s")
    wid = sid * 2 + cid

    @pl.loop(0, 32)
    def _(r):
        @pl.loop(0, D // 16)
        def _(c):
            tmp_v[r, pl.ds(c * 16, 16)] = jnp.zeros((16,), jnp.float32)

    pltpu.sync_copy(tmp_v, acc_a.at[pl.ds(sid * 32, 32)])
    pltpu.sync_copy(tmp_v, acc_b.at[pl.ds(sid * 32, 32)])
    plsc.subcore_barrier()

    nb_bond = 39 + jnp.where(wid < NB_BOND - 39 * NW, 1, 0)

    @pl.loop(0, nb_bond)
    def _(j):
        b = j * NW + wid
        row0 = b * BBLK
        pltpu.sync_copy(evidx_hbm.at[pl.ds(row0, BBLK)], idx_v)
        pltpu.sync_copy(bseg_hbm.at[pl.ds(row0, BBLK)], bseg_v)
        pltpu.sync_copy(bond_hbm.at[idx_v], rows_v)
        pltpu.sync_copy(rows_v, acc_b.at[bseg_v], add=True)

    nb_atom = 3 + jnp.where(wid < NB_ATOM - 3 * NW, 1, 0)

    @pl.loop(0, nb_atom)
    def _(j):
        b = j * NW + wid
        row0 = b * ABLK
        pltpu.sync_copy(aseg_hbm.at[pl.ds(row0, ABLK)], aseg_v)
        pltpu.sync_copy(atom_hbm.at[pl.ds(row0, ABLK)], arows_v)
        pltpu.sync_copy(arows_v, acc_a.at[aseg_v], add=True)

    plsc.subcore_barrier()

    pltpu.sync_copy(acc_a.at[pl.ds(sid * 32, 32)], tmp_v)
    pltpu.sync_copy(tmp_v, out_a_hbm.at[cid, pl.ds(sid * 32, 32)])
    pltpu.sync_copy(acc_b.at[pl.ds(sid * 32, 32)], tmp_v)
    pltpu.sync_copy(tmp_v, out_b_hbm.at[cid, pl.ds(sid * 32, 32)])


def _sc_pool(bond_feats, even_idx, b_ids, atom_feats, a_ids):
    mesh = plsc.VectorSubcoreMesh(core_axis_name="c", subcore_axis_name="s")
    f32 = jnp.float32
    kern = pl.kernel(
        _sc_pool_body,
        out_type=(
            jax.ShapeDtypeStruct((2, B, D), f32),
            jax.ShapeDtypeStruct((2, B, D), f32),
        ),
        mesh=mesh,
        scratch_types=[
            pltpu.VMEM_SHARED((B, D), f32),
            pltpu.VMEM_SHARED((B, D), f32),
            pltpu.VMEM((BBLK, D), f32),
            pltpu.VMEM((ABLK, D), f32),
            pltpu.VMEM((BBLK,), jnp.int32),
            pltpu.VMEM((BBLK,), jnp.int32),
            pltpu.VMEM((ABLK,), jnp.int32),
            pltpu.VMEM((32, D), f32),
        ],
    )
    return kern(bond_feats, even_idx, b_ids, atom_feats, a_ids)


def _combine_body(pa_ref, pb_ref, g_ref, out_ref):
    out_ref[:, 0:D] = pa_ref[0] + pa_ref[1]
    out_ref[:, D : 2 * D] = pb_ref[0] + pb_ref[1]
    out_ref[:, 2 * D : 3 * D] = g_ref[...]


def _combine(pa, pb, g):
    return pl.pallas_call(
        _combine_body,
        out_shape=jax.ShapeDtypeStruct((B, 3 * D), jnp.float32),
    )(pa, pb, g)


def kernel(atom_feats, bond_feats, global_feats, atom_segment_ids, bond_segment_ids):
    a_ids = atom_segment_ids.astype(jnp.int32)
    b_ids = bond_segment_ids.astype(jnp.int32)
    even_idx = jnp.arange(0, bond_feats.shape[0], 2, dtype=jnp.int32)
    pa, pb = _sc_pool(bond_feats, even_idx, b_ids, atom_feats, a_ids)
    return _combine(pa, pb, global_feats)


# speedup vs baseline: 13.0760x; 1.9407x over previous
import jax
import jax.numpy as jnp
from jax import lax
from jax.experimental import pallas as pl
from jax.experimental.pallas import tpu as pltpu
from jax.experimental.pallas import tpu_sc as plsc

B = 512
D = 128

N_ATOMS = 10000
N_BONDS = 160000

BBLK = 128
ABLK = 80
NB_BOND = N_BONDS // BBLK
NB_ATOM = N_ATOMS // ABLK
NW = 32
BOND_FLOOR = NB_BOND // NW
ATOM_FLOOR = NB_ATOM // NW
BOND_MAX = BOND_FLOOR + 1
ATOM_MAX = ATOM_FLOOR + 1


def _sc_pool_body(
    bond_hbm,
    evidx_hbm,
    bseg_hbm,
    atom_hbm,
    aseg_hbm,
    out_a_hbm,
    out_b_hbm,
    acc_a,
    acc_b,
    rows0,
    rows1,
    arows0,
    arows1,
    bidx_all,
    bseg_all,
    aseg_all,
    tmp_v,
    sem_pre,
    gsem0,
    gsem1,
):
    cid = lax.axis_index("c")
    sid = lax.axis_index("s")
    wid = sid * 2 + cid

    nb = BOND_FLOOR + jnp.where(wid < NB_BOND - BOND_FLOOR * NW, 1, 0)
    na = ATOM_FLOOR + jnp.where(wid < NB_ATOM - ATOM_FLOOR * NW, 1, 0)

    @pl.loop(0, nb)
    def _(j):
        row0 = (j * NW + wid) * BBLK
        pltpu.async_copy(evidx_hbm.at[pl.ds(row0, BBLK)], bidx_all.at[j], sem_pre)
        pltpu.async_copy(bseg_hbm.at[pl.ds(row0, BBLK)], bseg_all.at[j], sem_pre)

    @pl.loop(0, na)
    def _(j):
        row0 = (j * NW + wid) * ABLK
        pltpu.async_copy(aseg_hbm.at[pl.ds(row0, ABLK)], aseg_all.at[j], sem_pre)

    @pl.loop(0, 32)
    def _(r):
        @pl.loop(0, D // 16)
        def _(c):
            tmp_v[r, pl.ds(c * 16, 16)] = jnp.zeros((16,), jnp.float32)

    pltpu.sync_copy(tmp_v, acc_a.at[pl.ds(sid * 32, 32)])
    pltpu.sync_copy(tmp_v, acc_b.at[pl.ds(sid * 32, 32)])

    @pl.loop(0, nb)
    def _(j):
        pltpu.make_async_copy(evidx_hbm.at[pl.ds(0, BBLK)], bidx_all.at[0], sem_pre).wait()
        pltpu.make_async_copy(bseg_hbm.at[pl.ds(0, BBLK)], bseg_all.at[0], sem_pre).wait()

    @pl.loop(0, na)
    def _(j):
        pltpu.make_async_copy(aseg_hbm.at[pl.ds(0, ABLK)], aseg_all.at[0], sem_pre).wait()

    plsc.subcore_barrier()

    def bond_gather(j, buf, sem):
        pltpu.async_copy(bond_hbm.at[bidx_all.at[j]], buf, sem)

    def bond_step(j, buf, sem, nxt_buf, nxt_sem):
        @pl.when(j < nb)
        def _():
            @pl.when(j + 1 < nb)
            def _():
                bond_gather(j + 1, nxt_buf, nxt_sem)

            pltpu.make_async_copy(bond_hbm.at[bidx_all.at[j]], buf, sem).wait()
            pltpu.sync_copy(buf, acc_b.at[bseg_all.at[j]], add=True)

    bond_gather(0, rows0, gsem0)

    @pl.loop(0, BOND_MAX, step=2)
    def _(j):
        bond_step(j, rows0, gsem0, rows1, gsem1)
        bond_step(j + 1, rows1, gsem1, rows0, gsem0)

    def atom_gather(j, buf, sem):
        row0 = (j * NW + wid) * ABLK
        pltpu.async_copy(atom_hbm.at[pl.ds(row0, ABLK)], buf, sem)

    def atom_step(j, buf, sem, nxt_buf, nxt_sem):
        @pl.when(j < na)
        def _():
            @pl.when(j + 1 < na)
            def _():
                atom_gather(j + 1, nxt_buf, nxt_sem)

            pltpu.make_async_copy(atom_hbm.at[pl.ds(0, ABLK)], buf, sem).wait()
            pltpu.sync_copy(buf, acc_a.at[aseg_all.at[j]], add=True)

    atom_gather(0, arows0, gsem0)

    @pl.loop(0, ATOM_MAX, step=2)
    def _(j):
        atom_step(j, arows0, gsem0, arows1, gsem1)
        atom_step(j + 1, arows1, gsem1, arows0, gsem0)

    plsc.subcore_barrier()

    pltpu.sync_copy(acc_a.at[pl.ds(sid * 32, 32)], tmp_v)
    pltpu.sync_copy(tmp_v, out_a_hbm.at[cid, pl.ds(sid * 32, 32)])
    pltpu.sync_copy(acc_b.at[pl.ds(sid * 32, 32)], tmp_v)
    pltpu.sync_copy(tmp_v, out_b_hbm.at[cid, pl.ds(sid * 32, 32)])


def _sc_pool(bond_feats, even_idx, b_ids, atom_feats, a_ids):
    mesh = plsc.VectorSubcoreMesh(core_axis_name="c", subcore_axis_name="s")
    f32 = jnp.float32
    i32 = jnp.int32
    kern = pl.kernel(
        _sc_pool_body,
        out_type=(
            jax.ShapeDtypeStruct((2, B, D), f32),
            jax.ShapeDtypeStruct((2, B, D), f32),
        ),
        mesh=mesh,
        scratch_types=[
            pltpu.VMEM_SHARED((B, D), f32),
            pltpu.VMEM_SHARED((B, D), f32),
            pltpu.VMEM((BBLK, D), f32),
            pltpu.VMEM((BBLK, D), f32),
            pltpu.VMEM((ABLK, D), f32),
            pltpu.VMEM((ABLK, D), f32),
            pltpu.VMEM((BOND_MAX, BBLK), i32),
            pltpu.VMEM((BOND_MAX, BBLK), i32),
            pltpu.VMEM((ATOM_MAX, ABLK), i32),
            pltpu.VMEM((32, D), f32),
            pltpu.SemaphoreType.DMA,
            pltpu.SemaphoreType.DMA,
            pltpu.SemaphoreType.DMA,
        ],
    )
    return kern(bond_feats, even_idx, b_ids, atom_feats, a_ids)


def _combine_body(pa_ref, pb_ref, g_ref, out_ref):
    out_ref[:, 0:D] = pa_ref[0] + pa_ref[1]
    out_ref[:, D : 2 * D] = pb_ref[0] + pb_ref[1]
    out_ref[:, 2 * D : 3 * D] = g_ref[...]


def _combine(pa, pb, g):
    return pl.pallas_call(
        _combine_body,
        out_shape=jax.ShapeDtypeStruct((B, 3 * D), jnp.float32),
    )(pa, pb, g)


def kernel(atom_feats, bond_feats, global_feats, atom_segment_ids, bond_segment_ids):
    a_ids = atom_segment_ids.astype(jnp.int32)
    b_ids = bond_segment_ids.astype(jnp.int32)
    even_idx = jnp.arange(0, bond_feats.shape[0], 2, dtype=jnp.int32)
    pa, pb = _sc_pool(bond_feats, even_idx, b_ids, atom_feats, a_ids)
    return _combine(pa, pb, global_feats)
